# Initial kernel scaffold; baseline (speedup 1.0000x reference)
import jax
import jax.numpy as jnp
from jax.experimental import pallas as pl


def kernel(x, edge_index, W1, b1, g1, bt1, W2, b2, g2, bt2, W3, b3):
    raise NotImplementedError("write your pallas kernel here")


# speedup vs baseline: 13.1196x; 13.1196x over previous
import functools

import jax
import jax.numpy as jnp
from jax import lax
from jax.experimental import pallas as pl
from jax.experimental.pallas import tpu as pltpu
from jax.experimental.pallas import tpu_sc as plsc

N = 10000
E = 320000
D = 128
H = 128
NC = 2
NS = 16
NP = 10240
HH = 64

CA = 80
NCH_A = E // (NC * NS) // CA

CB = 80
NCH_B = E // NS // CB

_MESH = dict(core_axis_name="c", subcore_axis_name="s", num_cores=NC,
             num_subcores=NS)


def _sc_agg16_body(tbl_hbm, srcr_hbm, dstr_hbm, zer_hbm, out_hbm,
                   src_v, dst_v, rows_v, acc, sem):
    cid = lax.axis_index("c")
    sid = lax.axis_index("s")
    rpt = NP // NS
    r0 = sid * rpt
    pltpu.sync_copy(zer_hbm.at[pl.ds(r0, rpt)], acc.at[pl.ds(r0, rpt)])
    pltpu.sync_copy(srcr_hbm.at[cid, sid], src_v)
    pltpu.sync_copy(dstr_hbm.at[cid, sid], dst_v)
    plsc.subcore_barrier()

    def step(j, carry):
        pltpu.async_copy(tbl_hbm.at[src_v.at[j]], rows_v, sem).wait()
        pltpu.sync_copy(rows_v, acc.at[dst_v.at[j]], add=True)
        return carry

    lax.fori_loop(0, NCH_A, step, 0, unroll=False)
    plsc.subcore_barrier()
    pltpu.sync_copy(acc.at[pl.ds(r0, rpt)], out_hbm.at[cid, pl.ds(r0, rpt)])


@functools.lru_cache(maxsize=None)
def _sc_agg16():
    return pl.kernel(
        _sc_agg16_body,
        out_type=jax.ShapeDtypeStruct((NC, NP, 16), jnp.float32),
        mesh=plsc.VectorSubcoreMesh(**_MESH),
        scratch_types=[
            pltpu.VMEM((NCH_A, CA), jnp.int32),
            pltpu.VMEM((NCH_A, CA), jnp.int32),
            pltpu.VMEM((CA, 16), jnp.float32),
            pltpu.VMEM_SHARED((NP, 16), jnp.float32),
            pltpu.SemaphoreType.DMA,
        ],
        compiler_params=pltpu.CompilerParams(use_tc_tiling_on_sc=False),
    )


def _sc_agg_big_body(hsx_hbm, src2_hbm, dstp_hbm, zer_hbm, out_hbm,
                     src_v, dst_v, rows_v, acc, sem):
    cid = lax.axis_index("c")
    sid = lax.axis_index("s")
    rpt = NP // NS
    r0 = sid * rpt
    pltpu.sync_copy(zer_hbm.at[pl.ds(r0, rpt)], acc.at[pl.ds(r0, rpt)])
    pltpu.sync_copy(src2_hbm.at[cid, sid], src_v)
    pltpu.sync_copy(dstp_hbm.at[sid], dst_v)
    plsc.subcore_barrier()

    def step(j, carry):
        pltpu.async_copy(hsx_hbm.at[src_v.at[j]], rows_v, sem).wait()
        pltpu.sync_copy(rows_v, acc.at[dst_v.at[j]], add=True)
        return carry

    lax.fori_loop(0, NCH_B, step, 0, unroll=False)
    plsc.subcore_barrier()
    pltpu.sync_copy(acc.at[pl.ds(r0, rpt)], out_hbm.at[pl.ds(r0, rpt), cid])


@functools.lru_cache(maxsize=None)
def _sc_agg_big():
    return pl.kernel(
        _sc_agg_big_body,
        out_type=jax.ShapeDtypeStruct((NP, NC, HH), jnp.float32),
        mesh=plsc.VectorSubcoreMesh(**_MESH),
        scratch_types=[
            pltpu.VMEM((NCH_B, CB), jnp.int32),
            pltpu.VMEM((NCH_B, CB), jnp.int32),
            pltpu.VMEM((CB, HH), jnp.float32),
            pltpu.VMEM_SHARED((NP, HH), jnp.float32),
            pltpu.SemaphoreType.DMA,
        ],
        compiler_params=pltpu.CompilerParams(use_tc_tiling_on_sc=False),
    )


def _tc1_body(degp, x, w1, eim, dis_o, hs1_o, src2_o):
    d = degp[...]
    deg = d[0][:N, 0:1] + d[1][:N, 0:1] + 1.0
    dis = 1.0 / jnp.sqrt(deg)
    dis_o[...] = dis
    h = jnp.dot(x[...], w1[...], preferred_element_type=jnp.float32)
    hs1_o[...] = h * dis
    srcm = eim[0]
    src2_o[0] = srcm * 2
    src2_o[1] = srcm * 2 + 1


_tc1 = pl.pallas_call(
    _tc1_body,
    out_shape=[
        jax.ShapeDtypeStruct((N, 1), jnp.float32),
        jax.ShapeDtypeStruct((N, H), jnp.float32),
        jax.ShapeDtypeStruct((NC, E // 128, 128), jnp.int32),
    ],
)


def _tc_mid_body(agg, hs, dis, b, g, bt, wn, out, *, pad16):
    t = (agg[...] + hs[...]) * dis[...] + b[...][None, :]
    mu = jnp.mean(t, axis=0, keepdims=True)
    tc = t - mu
    var = jnp.mean(tc * tc, axis=0, keepdims=True)
    y = g[...][None, :] * tc / jnp.sqrt(var + 1e-5) + bt[...][None, :]
    z = jnp.where(y >= 0, y, 0.01 * y)
    hn = jnp.dot(z, wn[...], preferred_element_type=jnp.float32) * dis[...]
    if pad16:
        col = lax.broadcasted_iota(jnp.int32, (1, 16), 1)
        out[...] = jnp.where(col == 0, hn, 0.0)
    else:
        out[...] = hn


_tc_mid128 = pl.pallas_call(
    functools.partial(_tc_mid_body, pad16=False),
    out_shape=jax.ShapeDtypeStruct((N, H), jnp.float32),
)

_tc_mid16 = pl.pallas_call(
    functools.partial(_tc_mid_body, pad16=True),
    out_shape=jax.ShapeDtypeStruct((N, 16), jnp.float32),
)


def _tc3_body(aggp, hs3, dis, b3, out):
    a = aggp[...]
    s = a[0][:N, 0:1] + a[1][:N, 0:1] + hs3[:, 0:1]
    out[...] = s * dis[...] + b3[...]


_tc3 = pl.pallas_call(
    _tc3_body,
    out_shape=jax.ShapeDtypeStruct((N, 1), jnp.float32),
)


def kernel(x, edge_index, W1, b1, g1, bt1, W2, b2, g2, bt2, W3, b3):
    src = edge_index[0]
    dst = edge_index[1]
    src_r16 = src.reshape(NC, NS, NCH_A, CA)
    dst_r16 = dst.reshape(NC, NS, NCH_A, CA)
    dst_rb = dst.reshape(NS, NCH_B, CB)
    ei_m = edge_index.reshape(2, E // 128, 128)

    ones16 = jnp.ones((N, 16), jnp.float32)
    zeros16 = jnp.zeros((NP, 16), jnp.float32)
    zeros64 = jnp.zeros((NP, HH), jnp.float32)

    degp = _sc_agg16()(ones16, dst_r16, dst_r16, zeros16)

    dis, hs1, src2m = _tc1(degp, x, W1, ei_m)
    src2 = src2m.reshape(NC, NS, NCH_B, CB)

    agg1 = _sc_agg_big()(hs1.reshape(2 * N, HH), src2, dst_rb, zeros64)
    hs2 = _tc_mid128(agg1.reshape(NP, H)[:N], hs1, dis, b1, g1, bt1, W2)

    agg2 = _sc_agg_big()(hs2.reshape(2 * N, HH), src2, dst_rb, zeros64)
    hs3p = _tc_mid16(agg2.reshape(NP, H)[:N], hs2, dis, b2, g2, bt2, W3)

    agg3 = _sc_agg16()(hs3p, src_r16, dst_r16, zeros16)
    return _tc3(agg3, hs3p, dis, b3)
